# initial kernel scaffold (unmeasured)
import jax
import jax.numpy as jnp
from jax import lax
from jax.experimental import pallas as pl
from jax.experimental.pallas import tpu as pltpu

N_DEV = 4
M, K, N = 4096, 4096, 8192
BLK = M // N_DEV
CH = 256
CPB = BLK // CH


def _ar_body(partial_ref, out_ref, comm_ref, part_ref, send_sems, recv_sems,
             copy_sem):
    my = lax.axis_index("i")
    left = lax.rem(my - 1 + N_DEV, N_DEV)
    right = lax.rem(my + 1, N_DEV)

    barrier = pltpu.get_barrier_semaphore()
    for nbr in (left, right):
        pl.semaphore_signal(barrier, inc=1, device_id=(nbr,),
                            device_id_type=pl.DeviceIdType.MESH)
    pl.semaphore_wait(barrier, 2)

    def row0(block, c):
        return block * BLK + c * CH

    for c in range(CPB):
        ld = pltpu.make_async_copy(
            partial_ref.at[pl.ds(row0(my, c), CH), :], comm_ref.at[0],
            copy_sem)
        ld.start()
        ld.wait()
        for h in range(2 * (N_DEV - 1)):
            s_slot, r_slot = h % 2, (h + 1) % 2
            rdma = pltpu.make_async_remote_copy(
                src_ref=comm_ref.at[s_slot],
                dst_ref=comm_ref.at[r_slot],
                send_sem=send_sems.at[s_slot],
                recv_sem=recv_sems.at[r_slot],
                device_id=(right,),
                device_id_type=pl.DeviceIdType.MESH)
            rdma.start()
            rdma.wait()
            if h < N_DEV - 1:
                rb = lax.rem(my - h - 1 + N_DEV, N_DEV)
                ld = pltpu.make_async_copy(
                    partial_ref.at[pl.ds(row0(rb, c), CH), :], part_ref,
                    copy_sem)
                ld.start()
                ld.wait()
                comm_ref[r_slot] = comm_ref[r_slot] + part_ref[...]
                if h == N_DEV - 2:
                    st = pltpu.make_async_copy(
                        comm_ref.at[r_slot],
                        out_ref.at[pl.ds(row0(rb, c), CH), :], copy_sem)
                    st.start()
                    st.wait()
            else:
                gb = lax.rem(my - (h - (N_DEV - 1)) + N_DEV, N_DEV)
                st = pltpu.make_async_copy(
                    comm_ref.at[r_slot],
                    out_ref.at[pl.ds(row0(gb, c), CH), :], copy_sem)
                st.start()
                st.wait()


def kernel(x, w_mat, scale_x, scale_w):
    s = (scale_x[0] * scale_w[0]).astype(jnp.float32)
    partial = lax.dot_general(
        x.astype(jnp.bfloat16), w_mat.astype(jnp.bfloat16),
        (((1,), (0,)), ((), ())),
        preferred_element_type=jnp.float32) * s

    return pl.pallas_call(
        _ar_body,
        out_shape=jax.ShapeDtypeStruct((M, N), jnp.float32),
        in_specs=[pl.BlockSpec(memory_space=pltpu.ANY)],
        out_specs=pl.BlockSpec(memory_space=pltpu.ANY),
        scratch_shapes=[
            pltpu.VMEM((2, CH, N), jnp.float32),
            pltpu.VMEM((CH, N), jnp.float32),
            pltpu.SemaphoreType.DMA((2,)),
            pltpu.SemaphoreType.DMA((2,)),
            pltpu.SemaphoreType.DMA,
        ],
        compiler_params=pltpu.CompilerParams(collective_id=0),
    )(partial)


# baseline (device time: 2516383 ns/iter reference)
import jax
import jax.numpy as jnp
from jax import lax
from jax.experimental import pallas as pl
from jax.experimental.pallas import tpu as pltpu

N_DEV = 4
M, K, N = 4096, 4096, 8192
BLK = M // N_DEV
CH = 256
CPB = BLK // CH


def _ar_body(partial_ref, out_ref, comm_ref, part_ref, send_sems, recv_sems,
             copy_sem):
    my = lax.axis_index("i")
    left = lax.rem(my - 1 + N_DEV, N_DEV)
    right = lax.rem(my + 1, N_DEV)

    barrier = pltpu.get_barrier_semaphore()
    for nbr in (left, right):
        pl.semaphore_signal(barrier, inc=1, device_id=(nbr,),
                            device_id_type=pl.DeviceIdType.MESH)
    pl.semaphore_wait(barrier, 2)

    def row0(block, c):
        return block * BLK + c * CH

    for c in range(CPB):
        ld = pltpu.make_async_copy(
            partial_ref.at[pl.ds(row0(my, c), CH), :], comm_ref.at[0],
            copy_sem)
        ld.start()
        ld.wait()
        for h in range(2 * (N_DEV - 1)):
            s_slot, r_slot = h % 2, (h + 1) % 2
            rdma = pltpu.make_async_remote_copy(
                src_ref=comm_ref.at[s_slot],
                dst_ref=comm_ref.at[r_slot],
                send_sem=send_sems.at[s_slot],
                recv_sem=recv_sems.at[r_slot],
                device_id=(right,),
                device_id_type=pl.DeviceIdType.MESH)
            rdma.start()
            rdma.wait()
            if h < N_DEV - 1:
                rb = lax.rem(my - h - 1 + N_DEV, N_DEV)
                ld = pltpu.make_async_copy(
                    partial_ref.at[pl.ds(row0(rb, c), CH), :], part_ref,
                    copy_sem)
                ld.start()
                ld.wait()
                comm_ref[r_slot] = comm_ref[r_slot] + part_ref[...]
                if h == N_DEV - 2:
                    st = pltpu.make_async_copy(
                        comm_ref.at[r_slot],
                        out_ref.at[pl.ds(row0(rb, c), CH), :], copy_sem)
                    st.start()
                    st.wait()
            else:
                gb = lax.rem(my - (h - (N_DEV - 1)) + N_DEV, N_DEV)
                st = pltpu.make_async_copy(
                    comm_ref.at[r_slot],
                    out_ref.at[pl.ds(row0(gb, c), CH), :], copy_sem)
                st.start()
                st.wait()


def kernel(x, w_mat, scale_x, scale_w):
    s = (scale_x[0] * scale_w[0]).astype(jnp.float32)
    partial = lax.dot_general(
        x.astype(jnp.bfloat16), w_mat.astype(jnp.bfloat16),
        (((1,), (0,)), ((), ())),
        preferred_element_type=jnp.float32) * s

    return pl.pallas_call(
        _ar_body,
        out_shape=jax.ShapeDtypeStruct((M, N), jnp.float32),
        in_specs=[pl.BlockSpec(memory_space=pltpu.MemorySpace.HBM)],
        out_specs=pl.BlockSpec(memory_space=pltpu.MemorySpace.HBM),
        scratch_shapes=[
            pltpu.VMEM((2, CH, N), jnp.float32),
            pltpu.VMEM((CH, N), jnp.float32),
            pltpu.SemaphoreType.DMA((2,)),
            pltpu.SemaphoreType.DMA((2,)),
            pltpu.SemaphoreType.DMA,
        ],
        compiler_params=pltpu.CompilerParams(collective_id=0),
    )(partial)


# device time: 1333524 ns/iter; 1.8870x vs baseline; 1.8870x over previous
import jax
import jax.numpy as jnp
from jax import lax
from jax.experimental import pallas as pl
from jax.experimental.pallas import tpu as pltpu

N_DEV = 4
M, K, N = 4096, 4096, 8192
BLK = M // N_DEV
CH = 512
CPB = BLK // CH
HALF = N // 2
N_HOP = 2 * (N_DEV - 1)


def _ar_body(partial_ref, out_ref,
             comm_r, part_r, comm_l, part_l,
             send_r, recv_r, send_l, recv_l,
             seed_sems, part_sems, store_sems):
    my = lax.axis_index("i")
    left = lax.rem(my + N_DEV - 1, N_DEV)
    right = lax.rem(my + 1, N_DEV)

    barrier = pltpu.get_barrier_semaphore()
    for nbr in (left, right):
        pl.semaphore_signal(barrier, inc=1, device_id=(nbr,),
                            device_id_type=pl.DeviceIdType.MESH)
    pl.semaphore_wait(barrier, 2)

    rings = (
        dict(idx=0, nbr=right, sgn=-1, comm=comm_r, part=part_r,
             send=send_r, recv=recv_r, col0=0),
        dict(idx=1, nbr=left, sgn=+1, comm=comm_l, part=part_l,
             send=send_l, recv=recv_l, col0=HALF),
    )

    def block(b):
        return lax.rem(b + 2 * N_DEV, N_DEV)

    pending = {}

    def flush(key):
        st = pending.pop(key, None)
        if st is not None:
            st.wait()

    for c in range(CPB):
        for r in rings:
            flush((r["idx"], 0))
            sd = pltpu.make_async_copy(
                partial_ref.at[pl.ds(my * BLK + c * CH, CH),
                               pl.ds(r["col0"], HALF)],
                r["comm"].at[0], seed_sems.at[r["idx"]])
            sd.start()
            pending[("seed", r["idx"])] = sd
        for r in rings:
            pending.pop(("seed", r["idx"])).wait()

        for h in range(N_HOP):
            s_slot, r_slot = h % 2, (h + 1) % 2
            rdmas = []
            for r in rings:
                rdma = pltpu.make_async_remote_copy(
                    src_ref=r["comm"].at[s_slot],
                    dst_ref=r["comm"].at[r_slot],
                    send_sem=r["send"].at[s_slot],
                    recv_sem=r["recv"].at[r_slot],
                    device_id=(r["nbr"],),
                    device_id_type=pl.DeviceIdType.MESH)
                rdma.start()
                rdmas.append(rdma)
            loads = []
            if h < N_DEV - 1:
                for r in rings:
                    rb = block(my + r["sgn"] * (h + 1))
                    ld = pltpu.make_async_copy(
                        partial_ref.at[pl.ds(rb * BLK + c * CH, CH),
                                       pl.ds(r["col0"], HALF)],
                        r["part"], part_sems.at[r["idx"]])
                    ld.start()
                    loads.append(ld)
            for rdma in rdmas:
                rdma.wait()
            for ld in loads:
                ld.wait()
            for r in rings:
                if h < N_DEV - 1:
                    r["comm"][r_slot] = r["comm"][r_slot] + r["part"][...]
                    store_blk = block(my + r["sgn"] * (h + 1))
                    do_store = h == N_DEV - 2
                else:
                    store_blk = block(my + r["sgn"] * (h - (N_DEV - 1)))
                    do_store = True
                if do_store:
                    key = (r["idx"], r_slot)
                    flush(key)
                    st = pltpu.make_async_copy(
                        r["comm"].at[r_slot],
                        out_ref.at[pl.ds(store_blk * BLK + c * CH, CH),
                                   pl.ds(r["col0"], HALF)],
                        store_sems.at[r["idx"], r_slot])
                    st.start()
                    pending[key] = st

    for key in list(pending):
        flush(key)


def kernel(x, w_mat, scale_x, scale_w):
    s = (scale_x[0] * scale_w[0]).astype(jnp.float32)
    partial = lax.dot_general(
        x.astype(jnp.bfloat16), w_mat.astype(jnp.bfloat16),
        (((1,), (0,)), ((), ())),
        preferred_element_type=jnp.float32) * s

    return pl.pallas_call(
        _ar_body,
        out_shape=jax.ShapeDtypeStruct((M, N), jnp.float32),
        in_specs=[pl.BlockSpec(memory_space=pltpu.MemorySpace.HBM)],
        out_specs=pl.BlockSpec(memory_space=pltpu.MemorySpace.HBM),
        scratch_shapes=[
            pltpu.VMEM((2, CH, HALF), jnp.float32),
            pltpu.VMEM((CH, HALF), jnp.float32),
            pltpu.VMEM((2, CH, HALF), jnp.float32),
            pltpu.VMEM((CH, HALF), jnp.float32),
            pltpu.SemaphoreType.DMA((2,)),
            pltpu.SemaphoreType.DMA((2,)),
            pltpu.SemaphoreType.DMA((2,)),
            pltpu.SemaphoreType.DMA((2,)),
            pltpu.SemaphoreType.DMA((2,)),
            pltpu.SemaphoreType.DMA((2,)),
            pltpu.SemaphoreType.DMA((2, 2)),
        ],
        compiler_params=pltpu.CompilerParams(
            collective_id=0, vmem_limit_bytes=100 * 1024 * 1024),
    )(partial)


# device time: 780659 ns/iter; 3.2234x vs baseline; 1.7082x over previous
import jax
import jax.numpy as jnp
from jax import lax
from jax.experimental import pallas as pl
from jax.experimental.pallas import tpu as pltpu

N_DEV = 4
M, K, N = 4096, 4096, 8192
BLK = M // N_DEV
CH = 512
CPB = BLK // CH
HALF = N // 2
N_HOP = 2 * (N_DEV - 1)


def _ar_body(partial_ref, out_ref,
             comm_r, part_r, stage_r, comm_l, part_l, stage_l,
             send_r, recv_r, send_l, recv_l,
             seed_sems, part_sems, store_sems):
    my = lax.axis_index("i")
    left = lax.rem(my + N_DEV - 1, N_DEV)
    right = lax.rem(my + 1, N_DEV)

    barrier = pltpu.get_barrier_semaphore()
    for nbr in (left, right):
        pl.semaphore_signal(barrier, inc=1, device_id=(nbr,),
                            device_id_type=pl.DeviceIdType.MESH)
    pl.semaphore_wait(barrier, 2)

    rings = (
        dict(idx=0, nbr=right, sgn=-1, comm=comm_r, part=part_r,
             stage=stage_r, send=send_r, recv=recv_r, col0=0),
        dict(idx=1, nbr=left, sgn=+1, comm=comm_l, part=part_l,
             stage=stage_l, send=send_l, recv=recv_l, col0=HALF),
    )

    def block(b):
        return lax.rem(b + 2 * N_DEV, N_DEV)

    in_flight_store = {}

    def do_store(r, r_slot, store_blk, c):
        st = in_flight_store.pop(r["idx"], None)
        if st is not None:
            st.wait()
        r["stage"][...] = r["comm"][r_slot].astype(jnp.float32)
        st = pltpu.make_async_copy(
            r["stage"],
            out_ref.at[pl.ds(store_blk * BLK + c * CH, CH),
                       pl.ds(r["col0"], HALF)],
            store_sems.at[r["idx"]])
        st.start()
        in_flight_store[r["idx"]] = st

    for c in range(CPB):
        seeds = []
        for r in rings:
            sd = pltpu.make_async_copy(
                partial_ref.at[pl.ds(my * BLK + c * CH, CH),
                               pl.ds(r["col0"], HALF)],
                r["comm"].at[0], seed_sems.at[r["idx"]])
            sd.start()
            seeds.append(sd)
        for sd in seeds:
            sd.wait()

        deferred = []
        for h in range(N_HOP):
            s_slot, r_slot = h % 2, (h + 1) % 2
            rdmas = []
            for r in rings:
                rdma = pltpu.make_async_remote_copy(
                    src_ref=r["comm"].at[s_slot],
                    dst_ref=r["comm"].at[r_slot],
                    send_sem=r["send"].at[s_slot],
                    recv_sem=r["recv"].at[r_slot],
                    device_id=(r["nbr"],),
                    device_id_type=pl.DeviceIdType.MESH)
                rdma.start()
                rdmas.append(rdma)
            loads = []
            if h < N_DEV - 1:
                for r in rings:
                    rb = block(my + r["sgn"] * (h + 1))
                    ld = pltpu.make_async_copy(
                        partial_ref.at[pl.ds(rb * BLK + c * CH, CH),
                                       pl.ds(r["col0"], HALF)],
                        r["part"], part_sems.at[r["idx"]])
                    ld.start()
                    loads.append(ld)
            for args in deferred:
                do_store(*args, c)
            deferred = []
            for rdma in rdmas:
                rdma.wait()
            for ld in loads:
                ld.wait()
            for r in rings:
                if h < N_DEV - 1:
                    r["comm"][r_slot] = (
                        r["comm"][r_slot].astype(jnp.float32)
                        + r["part"][...].astype(jnp.float32)
                    ).astype(jnp.bfloat16)
                    if h == N_DEV - 2:
                        deferred.append(
                            (r, r_slot, block(my + r["sgn"] * (h + 1))))
                else:
                    deferred.append(
                        (r, r_slot, block(my + r["sgn"] * (h - (N_DEV - 1)))))
        for args in deferred:
            do_store(*args, c)

    for st in in_flight_store.values():
        st.wait()


def kernel(x, w_mat, scale_x, scale_w):
    s = (scale_x[0] * scale_w[0]).astype(jnp.float32)
    xs = (x.astype(jnp.float32) * s).astype(jnp.bfloat16)
    partial = lax.dot_general(
        xs, w_mat.astype(jnp.bfloat16),
        (((1,), (0,)), ((), ())),
        preferred_element_type=jnp.float32).astype(jnp.bfloat16)

    return pl.pallas_call(
        _ar_body,
        out_shape=jax.ShapeDtypeStruct((M, N), jnp.float32),
        in_specs=[pl.BlockSpec(memory_space=pltpu.MemorySpace.HBM)],
        out_specs=pl.BlockSpec(memory_space=pltpu.MemorySpace.HBM),
        scratch_shapes=[
            pltpu.VMEM((2, CH, HALF), jnp.bfloat16),
            pltpu.VMEM((CH, HALF), jnp.bfloat16),
            pltpu.VMEM((CH, HALF), jnp.float32),
            pltpu.VMEM((2, CH, HALF), jnp.bfloat16),
            pltpu.VMEM((CH, HALF), jnp.bfloat16),
            pltpu.VMEM((CH, HALF), jnp.float32),
            pltpu.SemaphoreType.DMA((2,)),
            pltpu.SemaphoreType.DMA((2,)),
            pltpu.SemaphoreType.DMA((2,)),
            pltpu.SemaphoreType.DMA((2,)),
            pltpu.SemaphoreType.DMA((2,)),
            pltpu.SemaphoreType.DMA((2,)),
            pltpu.SemaphoreType.DMA((2,)),
        ],
        compiler_params=pltpu.CompilerParams(
            collective_id=0, vmem_limit_bytes=100 * 1024 * 1024),
    )(partial)


# device time: 733646 ns/iter; 3.4300x vs baseline; 1.0641x over previous
import jax
import jax.numpy as jnp
from jax import lax
from jax.experimental import pallas as pl
from jax.experimental.pallas import tpu as pltpu

N_DEV = 4
M, K, N = 4096, 4096, 8192
KS = K // N_DEV
BLK = M // N_DEV
CH = 512
CPB = BLK // CH
HALF = N // 2
N_HOP = 2 * (N_DEV - 1)


def _ar_body(x_ref, w_ref, out_ref,
             comm_r, part_r, comm_l, part_l, stage,
             send_r, recv_r, send_l, recv_l, store_sem):
    my = lax.axis_index("i")
    left = lax.rem(my + N_DEV - 1, N_DEV)
    right = lax.rem(my + 1, N_DEV)

    barrier = pltpu.get_barrier_semaphore()
    for nbr in (left, right):
        pl.semaphore_signal(barrier, inc=1, device_id=(nbr,),
                            device_id_type=pl.DeviceIdType.MESH)
    pl.semaphore_wait(barrier, 2)

    rings = (
        dict(idx=0, nbr=right, sgn=-1, comm=comm_r, part=part_r,
             send=send_r, recv=recv_r, col0=0),
        dict(idx=1, nbr=left, sgn=+1, comm=comm_l, part=part_l,
             send=send_l, recv=recv_l, col0=HALF),
    )

    def block(b):
        return lax.rem(b + 2 * N_DEV, N_DEV)

    def gemm(blk, c, r):
        return jnp.dot(
            x_ref[pl.ds(blk * BLK + c * CH, CH), :],
            w_ref[:, pl.ds(r["col0"], HALF)],
            preferred_element_type=jnp.float32,
        ).astype(jnp.bfloat16)

    in_flight_store = []

    def do_store(r, r_slot, store_blk, c):
        while in_flight_store:
            in_flight_store.pop().wait()
        stage[...] = r["comm"][r_slot].astype(jnp.float32)
        st = pltpu.make_async_copy(
            stage,
            out_ref.at[pl.ds(store_blk * BLK + c * CH, CH),
                       pl.ds(r["col0"], HALF)],
            store_sem)
        st.start()
        in_flight_store.append(st)

    for c in range(CPB):
        for r in rings:
            r["comm"][0] = gemm(my, c, r)

        deferred = []
        for h in range(N_HOP):
            s_slot, r_slot = h % 2, (h + 1) % 2
            rdmas = []
            for r in rings:
                rdma = pltpu.make_async_remote_copy(
                    src_ref=r["comm"].at[s_slot],
                    dst_ref=r["comm"].at[r_slot],
                    send_sem=r["send"].at[s_slot],
                    recv_sem=r["recv"].at[r_slot],
                    device_id=(r["nbr"],),
                    device_id_type=pl.DeviceIdType.MESH)
                rdma.start()
                rdmas.append(rdma)
            if h < N_DEV - 1:
                for r in rings:
                    r["part"][...] = gemm(block(my + r["sgn"] * (h + 1)), c, r)
            for args in deferred:
                do_store(*args, c)
            deferred = []
            for rdma in rdmas:
                rdma.wait()
            for r in rings:
                if h < N_DEV - 1:
                    r["comm"][r_slot] = (
                        r["comm"][r_slot].astype(jnp.float32)
                        + r["part"][...].astype(jnp.float32)
                    ).astype(jnp.bfloat16)
                    if h == N_DEV - 2:
                        deferred.append(
                            (r, r_slot, block(my + r["sgn"] * (h + 1))))
                else:
                    deferred.append(
                        (r, r_slot, block(my + r["sgn"] * (h - (N_DEV - 1)))))
        for args in deferred:
            do_store(*args, c)

    while in_flight_store:
        in_flight_store.pop().wait()


def kernel(x, w_mat, scale_x, scale_w):
    s = (scale_x[0] * scale_w[0]).astype(jnp.float32)
    xs = (x.astype(jnp.float32) * s).astype(jnp.bfloat16)
    ws = w_mat.astype(jnp.bfloat16)

    return pl.pallas_call(
        _ar_body,
        out_shape=jax.ShapeDtypeStruct((M, N), jnp.float32),
        in_specs=[
            pl.BlockSpec(memory_space=pltpu.MemorySpace.VMEM),
            pl.BlockSpec(memory_space=pltpu.MemorySpace.VMEM),
        ],
        out_specs=pl.BlockSpec(memory_space=pltpu.MemorySpace.HBM),
        scratch_shapes=[
            pltpu.VMEM((2, CH, HALF), jnp.bfloat16),
            pltpu.VMEM((CH, HALF), jnp.bfloat16),
            pltpu.VMEM((2, CH, HALF), jnp.bfloat16),
            pltpu.VMEM((CH, HALF), jnp.bfloat16),
            pltpu.VMEM((CH, HALF), jnp.float32),
            pltpu.SemaphoreType.DMA((2,)),
            pltpu.SemaphoreType.DMA((2,)),
            pltpu.SemaphoreType.DMA((2,)),
            pltpu.SemaphoreType.DMA((2,)),
            pltpu.SemaphoreType.DMA,
        ],
        compiler_params=pltpu.CompilerParams(
            collective_id=0, vmem_limit_bytes=100 * 1024 * 1024),
    )(xs, ws)
